# baseline (speedup 1.0000x reference)
import functools

import jax
import jax.numpy as jnp
from jax import lax
from jax.experimental import pallas as pl
from jax.experimental.pallas import tpu as pltpu
from jax.experimental.pallas import tpu_sc as plsc

_NC = 2
_NS = 16
_NW = _NC * _NS
_L = 16

_CHUNK = 16
_P = 2
_NPAT = 2 ** _P


@functools.lru_cache(maxsize=None)
def _build(b: int, s: int, d: int):
    n_rows = b * s
    r_per_w = n_rows // _NW
    n_chunks = r_per_w // _CHUNK
    n_grp = _CHUNK // _P
    mesh = plsc.VectorSubcoreMesh(core_axis_name="c", subcore_axis_name="s")

    @functools.partial(
        pl.kernel,
        mesh=mesh,
        compiler_params=pltpu.CompilerParams(needs_layout_passes=False),
        out_type=jax.ShapeDtypeStruct((b, s, d), jnp.float32),
        scratch_types=[
            pltpu.VMEM((r_per_w,), jnp.int32),
            pltpu.VMEM((_NPAT * _P, d), jnp.float32),
            pltpu.VMEM_SHARED((_NPAT * _P, d), jnp.float32),
            pltpu.SemaphoreType.DMA,
            pltpu.SemaphoreType.DMA,
        ],
    )
    def emb(idx_hbm, table_hbm, out_hbm, idx_v, pat_v, pat_s, sem_v, sem_s):
        sid = lax.axis_index("s")
        wid = sid * _NC + lax.axis_index("c")
        base = wid * r_per_w
        bb = base // s
        srow = base - bb * s

        pltpu.sync_copy(idx_hbm.at[bb, pl.ds(srow, r_per_w)], idx_v)

        for p in range(_NPAT):
            for j in range(_P):
                bit = (p >> (_P - 1 - j)) & 1
                pltpu.sync_copy(
                    table_hbm.at[pl.ds(bit, 1)],
                    pat_v.at[pl.ds(p * _P + j, 1)],
                )

        @pl.when(sid < _NPAT)
        def _build_shared():
            for j in range(_P):
                bit = jnp.right_shift(sid, _P - 1 - j) & 1
                pltpu.sync_copy(
                    table_hbm.at[pl.ds(bit, 1)],
                    pat_s.at[pl.ds(sid * _P + j, 1)],
                )

        plsc.subcore_barrier()

        ci = lax.iota(jnp.int32, _L)
        wvec = jnp.left_shift(jnp.int32(1), (_P - 1) - (ci % _P))
        gid = ci // _P

        def fire(c, pat, sem):
            rv = idx_v[pl.ds(c * _CHUNK, _L)]
            wv = rv * wvec
            for g in range(n_grp):
                pg = jnp.sum(jnp.where(gid == g, wv, 0))
                row = srow + c * _CHUNK + g * _P
                pltpu.async_copy(
                    pat.at[pl.ds(pg * _P, _P)],
                    out_hbm.at[bb, pl.ds(row, _P)],
                    sem,
                )

        def drain(pat, sem):
            for _ in range(n_grp):
                pltpu.make_async_copy(
                    pat.at[pl.ds(0, _P)],
                    out_hbm.at[0, pl.ds(0, _P)],
                    sem,
                ).wait()

        def quad(q):
            c0 = 4 * q
            fire(c0, pat_v, sem_v)
            fire(c0 + 1, pat_v, sem_v)
            fire(c0 + 2, pat_s, sem_s)
            fire(c0 + 3, pat_v, sem_v)

        def drain_quad():
            for _ in range(3):
                drain(pat_v, sem_v)
            drain(pat_s, sem_s)

        quad(0)

        def body(q, carry):
            quad(q)
            drain_quad()
            return carry

        lax.fori_loop(1, n_chunks // 4, body, 0)
        drain_quad()

    return emb


def kernel(styles, table):
    b, s = styles.shape
    d = table.shape[1]
    return _build(b, s, d)(styles.astype(jnp.int32), table)


# speedup vs baseline: 1.0239x; 1.0239x over previous
import functools

import jax
import jax.numpy as jnp
from jax import lax
from jax.experimental import pallas as pl
from jax.experimental.pallas import tpu as pltpu
from jax.experimental.pallas import tpu_sc as plsc

_NC = 2
_NS = 16
_NW = _NC * _NS
_L = 16

_CHUNK = 16
_P = 2
_NPAT = 2 ** _P


@functools.lru_cache(maxsize=None)
def _build(b: int, s: int, d: int):
    n_rows = b * s
    r_per_w = n_rows // _NW
    n_chunks = r_per_w // _CHUNK
    n_grp = _CHUNK // _P
    mesh = plsc.VectorSubcoreMesh(core_axis_name="c", subcore_axis_name="s")

    @functools.partial(
        pl.kernel,
        mesh=mesh,
        compiler_params=pltpu.CompilerParams(needs_layout_passes=False),
        out_type=jax.ShapeDtypeStruct((b, s, d), jnp.float32),
        scratch_types=[
            pltpu.VMEM((r_per_w,), jnp.int32),
            pltpu.VMEM((_NPAT * _P, d), jnp.float32),
            pltpu.VMEM_SHARED((_NPAT * _P, d), jnp.float32),
            pltpu.SemaphoreType.DMA,
            pltpu.SemaphoreType.DMA,
        ],
    )
    def emb(idx_hbm, table_hbm, out_hbm, idx_v, pat_v, pat_s, sem_v, sem_s):
        sid = lax.axis_index("s")
        wid = sid * _NC + lax.axis_index("c")
        base = wid * r_per_w
        bb = base // s
        srow = base - bb * s

        pltpu.sync_copy(idx_hbm.at[bb, pl.ds(srow, r_per_w)], idx_v)

        for p in range(_NPAT):
            for j in range(_P):
                bit = (p >> (_P - 1 - j)) & 1
                pltpu.sync_copy(
                    table_hbm.at[pl.ds(bit, 1)],
                    pat_v.at[pl.ds(p * _P + j, 1)],
                )

        @pl.when(sid < _NPAT)
        def _build_shared():
            for j in range(_P):
                bit = jnp.right_shift(sid, _P - 1 - j) & 1
                pltpu.sync_copy(
                    table_hbm.at[pl.ds(bit, 1)],
                    pat_s.at[pl.ds(sid * _P + j, 1)],
                )

        plsc.subcore_barrier()

        ci = lax.iota(jnp.int32, _L)
        wvec = jnp.left_shift(jnp.int32(1), (_P - 1) - (ci % _P))
        gid = ci // _P

        def fire(c, pat, sem):
            rv = idx_v[pl.ds(c * _CHUNK, _L)]
            wv = rv * wvec
            for g in range(n_grp):
                pg = jnp.sum(jnp.where(gid == g, wv, 0))
                row = srow + c * _CHUNK + g * _P
                pltpu.async_copy(
                    pat.at[pl.ds(pg * _P, _P)],
                    out_hbm.at[bb, pl.ds(row, _P)],
                    sem,
                )

        def drain(pat, sem):
            for _ in range(n_grp):
                pltpu.make_async_copy(
                    pat.at[pl.ds(0, _P)],
                    out_hbm.at[0, pl.ds(0, _P)],
                    sem,
                ).wait()

        fire(0, pat_v, sem_v)
        fire(1, pat_s, sem_s)

        def body(p, carry):
            fire(2 * p, pat_v, sem_v)
            drain(pat_v, sem_v)
            fire(2 * p + 1, pat_s, sem_s)
            drain(pat_s, sem_s)
            return carry

        lax.fori_loop(1, n_chunks // 2, body, 0)
        drain(pat_v, sem_v)
        drain(pat_s, sem_s)

    return emb


def kernel(styles, table):
    b, s = styles.shape
    d = table.shape[1]
    return _build(b, s, d)(styles.astype(jnp.int32), table)
